# Initial kernel scaffold; baseline (speedup 1.0000x reference)
import jax
import jax.numpy as jnp
from jax.experimental import pallas as pl


def kernel(x, pool_src, pool_dst, up_src, up_dst, up_kernel, W, b):
    raise NotImplementedError("write your pallas kernel here")


# speedup vs baseline: 25.0652x; 25.0652x over previous
import functools

import jax
import jax.numpy as jnp
from jax import lax
from jax.experimental import pallas as pl
from jax.experimental.pallas import tpu as pltpu
from jax.experimental.pallas import tpu_sc as plsc

M_SEG = 25000
NC = 2
NS = 16
NW = NC * NS
LANES = 16
IDXW = 128
CHUNK = 8 * IDXW
RPC = CHUNK // IDXW


def _pad_to(n, m):
    return ((n + m - 1) // m) * m


def _pool_body(nrows_pw, mt, src2, dst2, x_hbm, sums_out, counts_out,
               sums_sh, srcbuf, dstbuf, rows, counts, gsem):
    c = lax.axis_index("c")
    s = lax.axis_index("s")
    w = c * NS + s
    mp = counts.shape[0]
    z16 = jnp.zeros((LANES,), jnp.float32)

    def zero_counts(i, carry):
        counts[pl.ds(i * LANES, LANES)] = z16
        return carry

    lax.fori_loop(0, mp // LANES, zero_counts, 0)

    def zero_rows(i, carry):
        rows[i, pl.ds(0, LANES)] = z16
        rows[i, pl.ds(LANES, LANES)] = z16
        return carry

    lax.fori_loop(0, CHUNK, zero_rows, 0)
    off = 0
    while off < mt:
        sz = min(CHUNK, mt - off)
        pltpu.sync_copy(rows.at[pl.ds(0, sz), :],
                        sums_sh.at[pl.ds(s * mt + off, sz), :])
        off += sz
    plsc.subcore_barrier()

    ones = jnp.full((LANES,), 1.0, jnp.float32)

    def chunk(j, carry):
        rb = w * nrows_pw + j * RPC
        pltpu.sync_copy(src2.at[pl.ds(rb, RPC), :], srcbuf)
        pltpu.sync_copy(dst2.at[pl.ds(rb, RPC), :], dstbuf)
        descs = [
            pltpu.async_copy(x_hbm.at[srcbuf.at[i]],
                             rows.at[pl.ds(i * IDXW, IDXW), :], gsem)
            for i in range(RPC)
        ]
        for d in descs:
            d.wait()
        for i in range(RPC):
            pltpu.sync_copy(rows.at[pl.ds(i * IDXW, IDXW), :],
                            sums_sh.at[dstbuf.at[i]], add=True)
        for i in range(RPC):
            for l in range(IDXW // LANES):
                dv = dstbuf[i, pl.ds(l * LANES, LANES)]
                plsc.addupdate_scatter(counts, [dv], ones)
        return carry

    lax.fori_loop(0, nrows_pw // RPC, chunk, 0)
    plsc.subcore_barrier()
    pltpu.sync_copy(sums_sh.at[pl.ds(s * mt, mt), :],
                    sums_out.at[c, pl.ds(s * mt, mt), :])
    pltpu.sync_copy(counts, counts_out.at[w])


def _up_body(nrows_pw, nt, kk, src2, kern2, dst2, p_hbm, up_out,
             up_sh, sbuf, kbuf, dbuf, gbuf, rows, gsem):
    c = lax.axis_index("c")
    s = lax.axis_index("s")
    w = c * NS + s
    base = s * nt
    zb = jnp.zeros((2 * LANES,), jnp.bfloat16)

    def zero_rows(i, carry):
        rows[i, :] = zb
        return carry

    lax.fori_loop(0, CHUNK, zero_rows, 0)
    off = 0
    while off < nt:
        sz = min(CHUNK, nt - off)
        pltpu.sync_copy(rows.at[pl.ds(0, sz), :],
                        up_sh.at[pl.ds(base + off, sz), :])
        off += sz
    plsc.subcore_barrier()

    def chunk(j, carry):
        rb = w * nrows_pw + j * RPC
        pltpu.sync_copy(src2.at[pl.ds(rb, RPC), :], sbuf)
        pltpu.sync_copy(kern2.at[pl.ds(rb, RPC), :], kbuf)
        pltpu.sync_copy(dst2.at[pl.ds(rb, RPC), :], dbuf)
        for i in range(RPC):
            for l in range(IDXW // LANES):
                sv = sbuf[i, pl.ds(l * LANES, LANES)]
                kv = kbuf[i, pl.ds(l * LANES, LANES)]
                gbuf[i, pl.ds(l * LANES, LANES)] = sv * kk + kv
        descs = [
            pltpu.async_copy(p_hbm.at[gbuf.at[i]],
                             rows.at[pl.ds(i * IDXW, IDXW), :], gsem)
            for i in range(RPC)
        ]
        for d in descs:
            d.wait()
        for i in range(RPC):
            pltpu.sync_copy(rows.at[pl.ds(i * IDXW, IDXW), :],
                            up_sh.at[dbuf.at[i]], add=True)
        return carry

    lax.fori_loop(0, nrows_pw // RPC, chunk, 0)
    plsc.subcore_barrier()
    off = 0
    while off < nt:
        sz = min(CHUNK, nt - off)
        pltpu.sync_copy(up_sh.at[pl.ds(base + off, sz), :],
                        up_out.at[c, pl.ds(base + off, sz), :])
        off += sz


def _dense_body(sums_ref, counts_ref, wf_ref, out_ref):
    sums = sums_ref[0] + sums_ref[1]
    cnt = jnp.sum(counts_ref[...], axis=0)
    pooled = sums / jnp.maximum(cnt, 1.0)[:, None]
    out_ref[...] = jnp.dot(pooled, wf_ref[...],
                           preferred_element_type=jnp.float32
                           ).astype(jnp.bfloat16)


def _final_body(x_ref, up_ref, b_ref, o_ref):
    up = up_ref[0].astype(jnp.float32) + up_ref[1].astype(jnp.float32)
    o_ref[...] = x_ref[...] - up - b_ref[...]


def kernel(x, pool_src, pool_dst, up_src, up_dst, up_kernel, W, b):
    n, ch = x.shape
    e = pool_src.shape[0]
    kk = W.shape[0]
    i32 = jnp.int32

    mp = _pad_to(M_SEG + 1, 8 * NS)
    np_ = _pad_to(n + 1, 8 * NS)
    mt = mp // NS
    nt = np_ // NS
    epad = _pad_to(e, NW * CHUNK)
    nrows_pw = epad // (NW * IDXW)
    pad = epad - e

    ps = jnp.concatenate([pool_src.astype(i32), jnp.zeros((pad,), i32)])
    pd = jnp.concatenate([pool_dst.astype(i32), jnp.full((pad,), M_SEG, i32)])
    us = jnp.concatenate([up_src.astype(i32), jnp.zeros((pad,), i32)])
    uk = jnp.concatenate([up_kernel.astype(i32), jnp.zeros((pad,), i32)])
    ud = jnp.concatenate([up_dst.astype(i32), jnp.full((pad,), n, i32)])
    src2 = ps.reshape(-1, IDXW)
    dst2 = pd.reshape(-1, IDXW)
    usrc2 = us.reshape(-1, IDXW)
    ukern2 = uk.reshape(-1, IDXW)
    udst2 = ud.reshape(-1, IDXW)

    mesh = plsc.VectorSubcoreMesh(core_axis_name="c", subcore_axis_name="s")

    sums_p, counts_p = pl.kernel(
        functools.partial(_pool_body, nrows_pw, mt),
        out_type=(jax.ShapeDtypeStruct((NC, mp, ch), jnp.float32),
                  jax.ShapeDtypeStruct((NW, mp), jnp.float32)),
        mesh=mesh,
        scratch_types=[
            pltpu.MemorySpace.VMEM_SHARED((mp, ch), jnp.float32),
            pltpu.VMEM((RPC, IDXW), i32),
            pltpu.VMEM((RPC, IDXW), i32),
            pltpu.VMEM((CHUNK, ch), jnp.float32),
            pltpu.VMEM((mp,), jnp.float32),
            pltpu.SemaphoreType.DMA,
        ],
        compiler_params=pltpu.CompilerParams(needs_layout_passes=False, use_tc_tiling_on_sc=False),
        name="sc_pool_segment_sum",
    )(src2, dst2, x)

    wf = W.transpose(1, 0, 2).reshape(ch, kk * ch)
    bm = mp // 14
    p_tab = pl.pallas_call(
        _dense_body,
        grid=(mp // bm,),
        in_specs=[
            pl.BlockSpec((NC, bm, ch), lambda j: (0, j, 0)),
            pl.BlockSpec((NW, bm), lambda j: (0, j)),
            pl.BlockSpec((ch, kk * ch), lambda j: (0, 0)),
        ],
        out_specs=pl.BlockSpec((bm, kk * ch), lambda j: (j, 0)),
        out_shape=jax.ShapeDtypeStruct((mp, kk * ch), jnp.bfloat16),
    )(sums_p, counts_p, wf)
    p_flat = p_tab.reshape(mp * kk, ch)

    up_p = pl.kernel(
        functools.partial(_up_body, nrows_pw, nt, kk),
        out_type=jax.ShapeDtypeStruct((NC, np_, ch), jnp.bfloat16),
        mesh=mesh,
        scratch_types=[
            pltpu.MemorySpace.VMEM_SHARED((np_, ch), jnp.bfloat16),
            pltpu.VMEM((RPC, IDXW), i32),
            pltpu.VMEM((RPC, IDXW), i32),
            pltpu.VMEM((RPC, IDXW), i32),
            pltpu.VMEM((RPC, IDXW), i32),
            pltpu.VMEM((CHUNK, ch), jnp.bfloat16),
            pltpu.SemaphoreType.DMA,
        ],
        compiler_params=pltpu.CompilerParams(needs_layout_passes=False, use_tc_tiling_on_sc=False),
        name="sc_upsample_scatter",
    )(usrc2, ukern2, udst2, p_flat)

    bn = 4000
    out = pl.pallas_call(
        _final_body,
        grid=(n // bn,),
        in_specs=[
            pl.BlockSpec((bn, ch), lambda j: (j, 0)),
            pl.BlockSpec((NC, bn, ch), lambda j: (0, j, 0)),
            pl.BlockSpec((1, ch), lambda j: (0, 0)),
        ],
        out_specs=pl.BlockSpec((bn, ch), lambda j: (j, 0)),
        out_shape=jax.ShapeDtypeStruct((n, ch), jnp.float32),
    )(x, up_p, b.reshape(1, ch))
    return out
